# baseline (speedup 1.0000x reference)
import jax
import jax.numpy as jnp
from jax import lax
from jax.experimental import pallas as pl
from jax.experimental.pallas import tpu as pltpu

_E = 16
_TEMP = 1.5
_CAP = 96
_H = 14
_W = 14


def _pool_kernel(x_ref, out_ref):
    x = x_ref[...]
    xt = jnp.swapaxes(x, 1, 2)
    hw = _H * _W

    def chain(rows):
        acc = None
        for b in range(_W):
            for a in rows:
                t = xt[:, a * _W + b, :]
                acc = t if acc is None else acc + t
        return acc

    s = chain(range(0, 7)) + chain(range(7, 14))
    out_ref[...] = s * (1.0 / hw)


def _stats_kernel(lr_ref, std_ref, z_ref, aux_ref, div_ref):
    lr = lr_ref[...]
    n, e = lr.shape
    mean = jnp.mean(lr)
    std_ref[...] = jnp.sqrt(jnp.mean((lr - mean) ** 2)).reshape(1, 1)

    l = jnp.clip(lr, -10.0, 10.0) / _TEMP

    m = jnp.max(l, axis=1, keepdims=True)
    se = jnp.sum(jnp.exp(l - m), axis=1, keepdims=True)
    lse = m + jnp.log(se)
    z_ref[...] = jnp.mean(lse * lse).reshape(1, 1)

    norm = jnp.maximum(jnp.sqrt(jnp.sum(l * l, axis=0, keepdims=True)),
                       1e-12)
    ln = l / norm
    rows = [jnp.sum(ln * ln[:, a:a + 1], axis=0, keepdims=True)
            for a in range(_E)]
    corr = jnp.concatenate(rows, axis=0)
    ia = lax.broadcasted_iota(jnp.int32, (_E, _E), 0)
    ib = lax.broadcasted_iota(jnp.int32, (_E, _E), 1)
    off = jnp.where(ia == ib, 0.0, corr)
    div_ref[...] = (jnp.sum(off * off) / (_E * (_E - 1))).reshape(1, 1)

    pe = jnp.exp(l - m)
    p = pe / se
    ep = jnp.max(p, axis=1, keepdims=True)
    it = lax.broadcasted_iota(jnp.int32, (n, e), 1)
    eidx = jnp.min(jnp.where(p == ep, it, e), axis=1, keepdims=True)
    onehot = (it == eidx).astype(jnp.float32)
    f = jnp.mean(onehot, axis=0, keepdims=True)
    pm = jnp.mean(p, axis=0, keepdims=True)
    aux_ref[...] = (jnp.sum(f * pm) * _E).reshape(1, 1)


def _route_kernel(ep_col_ref, eidx_col_ref, ep_row_ref, eidx_row_ref,
                  dispatch_ref, combine_ref):
    epc = ep_col_ref[...]
    eic = eidx_col_ref[...]
    epr = ep_row_ref[...]
    eir = eidx_row_ref[...]
    bi = epc.shape[0]
    n = epr.shape[1]

    jrow = lax.broadcasted_iota(jnp.int32, (1, n), 1)
    icol = (pl.program_id(0) * bi
            + lax.broadcasted_iota(jnp.int32, (bi, 1), 0))
    same = eir == eic
    beats = (epr > epc) | ((epr == epc) & (jrow < icol))
    rank = jnp.sum((same & beats).astype(jnp.int32), axis=1,
                   keepdims=True)
    within = rank < _CAP
    gate = jnp.where(within, epc, 0.0)

    e3 = lax.broadcasted_iota(jnp.int32, (bi, _E, _CAP), 1)
    c3 = lax.broadcasted_iota(jnp.int32, (bi, _E, _CAP), 2)
    hit = ((e3 == eic[:, :, None]) & (c3 == rank[:, :, None])
           & within[:, :, None])
    combine_ref[...] = jnp.where(hit, gate[:, :, None], 0.0)
    dispatch_ref[...] = hit


def kernel(X, W_gate, current_epoch):
    n, c, h, w = X.shape
    hw = h * w
    bt = 32
    Xr = X.reshape(n, c, hw)
    pooled = pl.pallas_call(
        _pool_kernel,
        grid=(n // bt,),
        in_specs=[
            pl.BlockSpec((bt, c, hw), lambda i: (i, 0, 0)),
        ],
        out_specs=pl.BlockSpec((bt, c), lambda i: (i, 0)),
        out_shape=jax.ShapeDtypeStruct((n, c), jnp.float32),
        compiler_params=pltpu.CompilerParams(
            dimension_semantics=("parallel",)),
    )(Xr)

    logits_raw = (pooled @ W_gate).astype(jnp.float32)
    logits = jnp.clip(logits_raw, -10.0, 10.0) / _TEMP
    probs = jax.nn.softmax(logits, axis=1)
    eidx = jnp.argmax(probs, axis=1).astype(jnp.int32)
    ep = jnp.take_along_axis(probs, eidx[:, None], axis=1)

    scalar = jax.ShapeDtypeStruct((1, 1), jnp.float32)
    stdv, z, aux, div = pl.pallas_call(
        _stats_kernel,
        out_shape=(scalar, scalar, scalar, scalar),
    )(logits_raw)

    ep_row = ep.reshape(1, n)
    eidx_row = eidx.reshape(1, n)
    eidx_col = eidx.reshape(n, 1)

    bi = 256
    dispatch, combine = pl.pallas_call(
        _route_kernel,
        grid=(n // bi,),
        in_specs=[
            pl.BlockSpec((bi, 1), lambda i: (i, 0)),
            pl.BlockSpec((bi, 1), lambda i: (i, 0)),
            pl.BlockSpec((1, n), lambda i: (0, 0)),
            pl.BlockSpec((1, n), lambda i: (0, 0)),
        ],
        out_specs=(
            pl.BlockSpec((bi, _E, _CAP), lambda i: (i, 0, 0)),
            pl.BlockSpec((bi, _E, _CAP), lambda i: (i, 0, 0)),
        ),
        out_shape=(
            jax.ShapeDtypeStruct((n, _E, _CAP), jnp.bool_),
            jax.ShapeDtypeStruct((n, _E, _CAP), jnp.float32),
        ),
        compiler_params=pltpu.CompilerParams(
            dimension_semantics=("arbitrary",)),
    )(ep, eidx_col, ep_row, eidx_row)

    return (dispatch, combine, z[0, 0], aux[0, 0], div[0, 0],
            stdv[0, 0], logits)


# speedup vs baseline: 1.4089x; 1.4089x over previous
import jax
import jax.numpy as jnp
from jax import lax
from jax.experimental import pallas as pl
from jax.experimental.pallas import tpu as pltpu

_E = 16
_TEMP = 1.5
_CAP = 96
_H = 14
_W = 14


def _pool_kernel(x_ref, out_ref):
    hw = _H * _W

    def chain(rows):
        acc = None
        for b in range(_W):
            for a in rows:
                t = x_ref[:, a, b, :]
                acc = t if acc is None else acc + t
        return acc

    s = chain(range(0, 7)) + chain(range(7, 14))
    out_ref[...] = s * (1.0 / hw)


def _stats_kernel(lr_ref, std_ref, z_ref, aux_ref, div_ref):
    lr = lr_ref[...]
    n, e = lr.shape
    mean = jnp.mean(lr)
    std_ref[...] = jnp.sqrt(jnp.mean((lr - mean) ** 2)).reshape(1, 1)

    l = jnp.clip(lr, -10.0, 10.0) / _TEMP

    m = jnp.max(l, axis=1, keepdims=True)
    se = jnp.sum(jnp.exp(l - m), axis=1, keepdims=True)
    lse = m + jnp.log(se)
    z_ref[...] = jnp.mean(lse * lse).reshape(1, 1)

    norm = jnp.maximum(jnp.sqrt(jnp.sum(l * l, axis=0, keepdims=True)),
                       1e-12)
    ln = l / norm
    rows = [jnp.sum(ln * ln[:, a:a + 1], axis=0, keepdims=True)
            for a in range(_E)]
    corr = jnp.concatenate(rows, axis=0)
    ia = lax.broadcasted_iota(jnp.int32, (_E, _E), 0)
    ib = lax.broadcasted_iota(jnp.int32, (_E, _E), 1)
    off = jnp.where(ia == ib, 0.0, corr)
    div_ref[...] = (jnp.sum(off * off) / (_E * (_E - 1))).reshape(1, 1)

    pe = jnp.exp(l - m)
    p = pe / se
    ep = jnp.max(p, axis=1, keepdims=True)
    it = lax.broadcasted_iota(jnp.int32, (n, e), 1)
    eidx = jnp.min(jnp.where(p == ep, it, e), axis=1, keepdims=True)
    onehot = (it == eidx).astype(jnp.float32)
    f = jnp.mean(onehot, axis=0, keepdims=True)
    pm = jnp.mean(p, axis=0, keepdims=True)
    aux_ref[...] = (jnp.sum(f * pm) * _E).reshape(1, 1)


def _route_kernel(ep_col_ref, eidx_col_ref, ep_row_ref, eidx_row_ref,
                  dispatch_ref, combine_ref):
    epc = ep_col_ref[...]
    eic = eidx_col_ref[...]
    epr = ep_row_ref[...]
    eir = eidx_row_ref[...]
    bi = epc.shape[0]
    n = epr.shape[1]

    jrow = lax.broadcasted_iota(jnp.int32, (1, n), 1)
    icol = (pl.program_id(0) * bi
            + lax.broadcasted_iota(jnp.int32, (bi, 1), 0))
    same = eir == eic
    beats = (epr > epc) | ((epr == epc) & (jrow < icol))
    rank = jnp.sum((same & beats).astype(jnp.int32), axis=1,
                   keepdims=True)
    within = rank < _CAP
    gate = jnp.where(within, epc, 0.0)

    e3 = lax.broadcasted_iota(jnp.int32, (bi, _E, _CAP), 1)
    c3 = lax.broadcasted_iota(jnp.int32, (bi, _E, _CAP), 2)
    hit = ((e3 == eic[:, :, None]) & (c3 == rank[:, :, None])
           & within[:, :, None])
    combine_ref[...] = jnp.where(hit, gate[:, :, None], 0.0)
    dispatch_ref[...] = hit


def kernel(X, W_gate, current_epoch):
    n, c, h, w = X.shape
    bt = 32
    Xp = jnp.transpose(X, (0, 2, 3, 1))
    pooled = pl.pallas_call(
        _pool_kernel,
        grid=(n // bt,),
        in_specs=[
            pl.BlockSpec((bt, h, w, c), lambda i: (i, 0, 0, 0)),
        ],
        out_specs=pl.BlockSpec((bt, c), lambda i: (i, 0)),
        out_shape=jax.ShapeDtypeStruct((n, c), jnp.float32),
        compiler_params=pltpu.CompilerParams(
            dimension_semantics=("parallel",)),
    )(Xp)

    logits_raw = (pooled @ W_gate).astype(jnp.float32)
    logits = jnp.clip(logits_raw, -10.0, 10.0) / _TEMP
    probs = jax.nn.softmax(logits, axis=1)
    eidx = jnp.argmax(probs, axis=1).astype(jnp.int32)
    ep = jnp.take_along_axis(probs, eidx[:, None], axis=1)

    scalar = jax.ShapeDtypeStruct((1, 1), jnp.float32)
    stdv, z, aux, div = pl.pallas_call(
        _stats_kernel,
        out_shape=(scalar, scalar, scalar, scalar),
    )(logits_raw)

    ep_row = ep.reshape(1, n)
    eidx_row = eidx.reshape(1, n)
    eidx_col = eidx.reshape(n, 1)

    bi = 256
    dispatch, combine = pl.pallas_call(
        _route_kernel,
        grid=(n // bi,),
        in_specs=[
            pl.BlockSpec((bi, 1), lambda i: (i, 0)),
            pl.BlockSpec((bi, 1), lambda i: (i, 0)),
            pl.BlockSpec((1, n), lambda i: (0, 0)),
            pl.BlockSpec((1, n), lambda i: (0, 0)),
        ],
        out_specs=(
            pl.BlockSpec((bi, _E, _CAP), lambda i: (i, 0, 0)),
            pl.BlockSpec((bi, _E, _CAP), lambda i: (i, 0, 0)),
        ),
        out_shape=(
            jax.ShapeDtypeStruct((n, _E, _CAP), jnp.bool_),
            jax.ShapeDtypeStruct((n, _E, _CAP), jnp.float32),
        ),
        compiler_params=pltpu.CompilerParams(
            dimension_semantics=("arbitrary",)),
    )(ep, eidx_col, ep_row, eidx_row)

    return (dispatch, combine, z[0, 0], aux[0, 0], div[0, 0],
            stdv[0, 0], logits)


# speedup vs baseline: 3.8578x; 2.7382x over previous
import jax
import jax.numpy as jnp
from jax import lax
from jax.experimental import pallas as pl
from jax.experimental.pallas import tpu as pltpu

_E = 16
_TEMP = 1.5
_CAP = 96
_H = 14
_W = 14


def _pool_kernel(x_ref, out_ref):
    hw = _H * _W

    def chain(rows):
        acc = None
        for b in range(_W):
            for a in rows:
                t = x_ref[a, b, :, :]
                acc = t if acc is None else acc + t
        return acc

    s = chain(range(0, 7)) + chain(range(7, 14))
    out_ref[...] = s * (1.0 / hw)


def _stats_kernel(lr_ref, std_ref, z_ref, aux_ref, div_ref):
    lr = lr_ref[...]
    n, e = lr.shape
    mean = jnp.mean(lr)
    std_ref[...] = jnp.sqrt(jnp.mean((lr - mean) ** 2)).reshape(1, 1)

    l = jnp.clip(lr, -10.0, 10.0) / _TEMP

    m = jnp.max(l, axis=1, keepdims=True)
    se = jnp.sum(jnp.exp(l - m), axis=1, keepdims=True)
    lse = m + jnp.log(se)
    z_ref[...] = jnp.mean(lse * lse).reshape(1, 1)

    norm = jnp.maximum(jnp.sqrt(jnp.sum(l * l, axis=0, keepdims=True)),
                       1e-12)
    ln = l / norm
    rows = [jnp.sum(ln * ln[:, a:a + 1], axis=0, keepdims=True)
            for a in range(_E)]
    corr = jnp.concatenate(rows, axis=0)
    ia = lax.broadcasted_iota(jnp.int32, (_E, _E), 0)
    ib = lax.broadcasted_iota(jnp.int32, (_E, _E), 1)
    off = jnp.where(ia == ib, 0.0, corr)
    div_ref[...] = (jnp.sum(off * off) / (_E * (_E - 1))).reshape(1, 1)

    pe = jnp.exp(l - m)
    p = pe / se
    ep = jnp.max(p, axis=1, keepdims=True)
    it = lax.broadcasted_iota(jnp.int32, (n, e), 1)
    eidx = jnp.min(jnp.where(p == ep, it, e), axis=1, keepdims=True)
    onehot = (it == eidx).astype(jnp.float32)
    f = jnp.mean(onehot, axis=0, keepdims=True)
    pm = jnp.mean(p, axis=0, keepdims=True)
    aux_ref[...] = (jnp.sum(f * pm) * _E).reshape(1, 1)


def _route_kernel(ep_col_ref, eidx_col_ref, ep_row_ref, eidx_row_ref,
                  dispatch_ref, combine_ref):
    epc = ep_col_ref[...]
    eic = eidx_col_ref[...]
    epr = ep_row_ref[...]
    eir = eidx_row_ref[...]
    bi = epc.shape[0]
    n = epr.shape[1]

    jrow = lax.broadcasted_iota(jnp.int32, (1, n), 1)
    icol = (pl.program_id(0) * bi
            + lax.broadcasted_iota(jnp.int32, (bi, 1), 0))
    same = eir == eic
    beats = (epr > epc) | ((epr == epc) & (jrow < icol))
    rank = jnp.sum((same & beats).astype(jnp.int32), axis=1,
                   keepdims=True)
    within = rank < _CAP
    gate = jnp.where(within, epc, 0.0)

    e3 = lax.broadcasted_iota(jnp.int32, (bi, _E, _CAP), 1)
    c3 = lax.broadcasted_iota(jnp.int32, (bi, _E, _CAP), 2)
    hit = ((e3 == eic[:, :, None]) & (c3 == rank[:, :, None])
           & within[:, :, None])
    combine_ref[...] = jnp.where(hit, gate[:, :, None], 0.0)
    dispatch_ref[...] = hit


def kernel(X, W_gate, current_epoch):
    n, c, h, w = X.shape
    bt = 64
    Xp = jnp.transpose(X, (2, 3, 0, 1))
    pooled = pl.pallas_call(
        _pool_kernel,
        grid=(n // bt,),
        in_specs=[
            pl.BlockSpec((h, w, bt, c), lambda i: (0, 0, i, 0)),
        ],
        out_specs=pl.BlockSpec((bt, c), lambda i: (i, 0)),
        out_shape=jax.ShapeDtypeStruct((n, c), jnp.float32),
        compiler_params=pltpu.CompilerParams(
            dimension_semantics=("parallel",)),
    )(Xp)

    logits_raw = (pooled @ W_gate).astype(jnp.float32)
    logits = jnp.clip(logits_raw, -10.0, 10.0) / _TEMP
    probs = jax.nn.softmax(logits, axis=1)
    eidx = jnp.argmax(probs, axis=1).astype(jnp.int32)
    ep = jnp.take_along_axis(probs, eidx[:, None], axis=1)

    scalar = jax.ShapeDtypeStruct((1, 1), jnp.float32)
    stdv, z, aux, div = pl.pallas_call(
        _stats_kernel,
        out_shape=(scalar, scalar, scalar, scalar),
    )(logits_raw)

    ep_row = ep.reshape(1, n)
    eidx_row = eidx.reshape(1, n)
    eidx_col = eidx.reshape(n, 1)

    bi = 256
    dispatch, combine = pl.pallas_call(
        _route_kernel,
        grid=(n // bi,),
        in_specs=[
            pl.BlockSpec((bi, 1), lambda i: (i, 0)),
            pl.BlockSpec((bi, 1), lambda i: (i, 0)),
            pl.BlockSpec((1, n), lambda i: (0, 0)),
            pl.BlockSpec((1, n), lambda i: (0, 0)),
        ],
        out_specs=(
            pl.BlockSpec((bi, _E, _CAP), lambda i: (i, 0, 0)),
            pl.BlockSpec((bi, _E, _CAP), lambda i: (i, 0, 0)),
        ),
        out_shape=(
            jax.ShapeDtypeStruct((n, _E, _CAP), jnp.bool_),
            jax.ShapeDtypeStruct((n, _E, _CAP), jnp.float32),
        ),
        compiler_params=pltpu.CompilerParams(
            dimension_semantics=("arbitrary",)),
    )(ep, eidx_col, ep_row, eidx_row)

    return (dispatch, combine, z[0, 0], aux[0, 0], div[0, 0],
            stdv[0, 0], logits)


# speedup vs baseline: 4.3613x; 1.1305x over previous
import jax
import jax.numpy as jnp
from jax import lax
from jax.experimental import pallas as pl
from jax.experimental.pallas import tpu as pltpu

_E = 16
_TEMP = 1.5
_CAP = 96
_H = 14
_W = 14


def _pool_kernel(x_ref, out_ref):
    hw = _H * _W

    def chain(rows):
        acc = None
        for b in range(_W):
            for a in rows:
                t = x_ref[a, b, :, :]
                acc = t if acc is None else acc + t
        return acc

    s = chain(range(0, 7)) + chain(range(7, 14))
    out_ref[...] = s * (1.0 / hw)


def _stats_kernel(lr_ref, std_ref, z_ref, aux_ref, div_ref):
    lr = lr_ref[...]
    n, e = lr.shape
    mean = jnp.mean(lr)
    std_ref[...] = jnp.sqrt(jnp.mean((lr - mean) ** 2)).reshape(1, 1)

    l = jnp.clip(lr, -10.0, 10.0) / _TEMP

    m = jnp.max(l, axis=1, keepdims=True)
    se = jnp.sum(jnp.exp(l - m), axis=1, keepdims=True)
    lse = m + jnp.log(se)
    z_ref[...] = jnp.mean(lse * lse).reshape(1, 1)

    norm = jnp.maximum(jnp.sqrt(jnp.sum(l * l, axis=0, keepdims=True)),
                       1e-12)
    ln = l / norm
    rows = [jnp.sum(ln * ln[:, a:a + 1], axis=0, keepdims=True)
            for a in range(_E)]
    corr = jnp.concatenate(rows, axis=0)
    ia = lax.broadcasted_iota(jnp.int32, (_E, _E), 0)
    ib = lax.broadcasted_iota(jnp.int32, (_E, _E), 1)
    off = jnp.where(ia == ib, 0.0, corr)
    div_ref[...] = (jnp.sum(off * off) / (_E * (_E - 1))).reshape(1, 1)

    pe = jnp.exp(l - m)
    p = pe / se
    ep = jnp.max(p, axis=1, keepdims=True)
    it = lax.broadcasted_iota(jnp.int32, (n, e), 1)
    eidx = jnp.min(jnp.where(p == ep, it, e), axis=1, keepdims=True)
    onehot = (it == eidx).astype(jnp.float32)
    f = jnp.mean(onehot, axis=0, keepdims=True)
    pm = jnp.mean(p, axis=0, keepdims=True)
    aux_ref[...] = (jnp.sum(f * pm) * _E).reshape(1, 1)


def _route_kernel(ep_col_ref, eidx_col_ref, ep_row_ref, eidx_row_ref,
                  combine_ref):
    epc = ep_col_ref[...]
    eic = eidx_col_ref[...]
    epr = ep_row_ref[...]
    eir = eidx_row_ref[...]
    n = epc.shape[0]
    bi = epr.shape[1]

    jcol = lax.broadcasted_iota(jnp.int32, (n, 1), 0)
    irow = (pl.program_id(0) * bi
            + lax.broadcasted_iota(jnp.int32, (1, bi), 1))
    same = eic == eir
    beats = (epc > epr) | ((epc == epr) & (jcol < irow))
    rank = jnp.sum((same & beats).astype(jnp.int32), axis=0,
                   keepdims=True)

    e3 = lax.broadcasted_iota(jnp.int32, (_E, _CAP, bi), 0)
    c3 = lax.broadcasted_iota(jnp.int32, (_E, _CAP, bi), 1)
    hit = (e3 == eir[None, :, :]) & (c3 == rank[None, :, :])
    combine_ref[...] = jnp.where(hit, epr[None, :, :], 0.0)


def kernel(X, W_gate, current_epoch):
    n, c, h, w = X.shape
    bt = 64
    Xp = jnp.transpose(X, (2, 3, 0, 1))
    pooled = pl.pallas_call(
        _pool_kernel,
        grid=(n // bt,),
        in_specs=[
            pl.BlockSpec((h, w, bt, c), lambda i: (0, 0, i, 0)),
        ],
        out_specs=pl.BlockSpec((bt, c), lambda i: (i, 0)),
        out_shape=jax.ShapeDtypeStruct((n, c), jnp.float32),
        compiler_params=pltpu.CompilerParams(
            dimension_semantics=("parallel",)),
    )(Xp)

    logits_raw = (pooled @ W_gate).astype(jnp.float32)
    logits = jnp.clip(logits_raw, -10.0, 10.0) / _TEMP
    probs = jax.nn.softmax(logits, axis=1)
    eidx = jnp.argmax(probs, axis=1).astype(jnp.int32)
    ep = jnp.take_along_axis(probs, eidx[:, None], axis=1)

    scalar = jax.ShapeDtypeStruct((1, 1), jnp.float32)
    stdv, z, aux, div = pl.pallas_call(
        _stats_kernel,
        out_shape=(scalar, scalar, scalar, scalar),
    )(logits_raw)

    ep_row = ep.reshape(1, n)
    eidx_row = eidx.reshape(1, n)
    eidx_col = eidx.reshape(n, 1)

    bi = 256
    combine_t = pl.pallas_call(
        _route_kernel,
        grid=(n // bi,),
        in_specs=[
            pl.BlockSpec((n, 1), lambda i: (0, 0)),
            pl.BlockSpec((n, 1), lambda i: (0, 0)),
            pl.BlockSpec((1, bi), lambda i: (0, i)),
            pl.BlockSpec((1, bi), lambda i: (0, i)),
        ],
        out_specs=pl.BlockSpec((_E, _CAP, bi), lambda i: (0, 0, i)),
        out_shape=jax.ShapeDtypeStruct((_E, _CAP, n), jnp.float32),
        compiler_params=pltpu.CompilerParams(
            dimension_semantics=("arbitrary",)),
    )(ep, eidx_col, ep_row, eidx_row)

    combine = jnp.transpose(combine_t, (2, 0, 1))
    dispatch = combine > 0

    return (dispatch, combine, z[0, 0], aux[0, 0], div[0, 0],
            stdv[0, 0], logits)


# speedup vs baseline: 4.8903x; 1.1213x over previous
import jax
import jax.numpy as jnp
from jax import lax
from jax.experimental import pallas as pl
from jax.experimental.pallas import tpu as pltpu

_E = 16
_TEMP = 1.5
_CAP = 96
_H = 14
_W = 14


def _pool_kernel(x_ref, out_ref):
    hw = _H * _W

    def chain(rows):
        acc = None
        for b in range(_W):
            for a in rows:
                t = x_ref[a, b, :, :]
                acc = t if acc is None else acc + t
        return acc

    s = chain(range(0, 7)) + chain(range(7, 14))
    out_ref[...] = s * (1.0 / hw)


def _stats_kernel(lr_ref, std_ref, z_ref, aux_ref, div_ref):
    lr = lr_ref[...]
    e, n = lr.shape
    mean = jnp.mean(lr)
    std_ref[...] = jnp.sqrt(jnp.mean((lr - mean) ** 2)).reshape(1, 1)

    l = jnp.clip(lr, -10.0, 10.0) / _TEMP

    m = jnp.max(l, axis=0, keepdims=True)
    se = jnp.sum(jnp.exp(l - m), axis=0, keepdims=True)
    lse = m + jnp.log(se)
    z_ref[...] = jnp.mean(lse * lse).reshape(1, 1)

    norm = jnp.maximum(jnp.sqrt(jnp.sum(l * l, axis=1, keepdims=True)),
                       1e-12)
    ln = l / norm
    cols = [jnp.sum(ln * ln[a:a + 1, :], axis=1, keepdims=True)
            for a in range(_E)]
    corr = jnp.concatenate(cols, axis=1)
    ia = lax.broadcasted_iota(jnp.int32, (_E, _E), 0)
    ib = lax.broadcasted_iota(jnp.int32, (_E, _E), 1)
    off = jnp.where(ia == ib, 0.0, corr)
    div_ref[...] = (jnp.sum(off * off) / (_E * (_E - 1))).reshape(1, 1)

    pe = jnp.exp(l - m)
    p = pe / se
    ep = jnp.max(p, axis=0, keepdims=True)
    it = lax.broadcasted_iota(jnp.int32, (e, n), 0)
    eidx = jnp.min(jnp.where(p == ep, it, e), axis=0, keepdims=True)
    onehot = (it == eidx).astype(jnp.float32)
    f = jnp.mean(onehot, axis=1, keepdims=True)
    pm = jnp.mean(p, axis=1, keepdims=True)
    aux_ref[...] = (jnp.sum(f * pm) * _E).reshape(1, 1)


def _route_kernel(ep_col_ref, eidx_col_ref, ep_row_ref, eidx_row_ref,
                  combine_ref):
    epc = ep_col_ref[...]
    eic = eidx_col_ref[...]
    epr = ep_row_ref[...]
    eir = eidx_row_ref[...]
    n = epc.shape[0]
    bi = epr.shape[1]

    jcol = lax.broadcasted_iota(jnp.int32, (n, 1), 0)
    irow = (pl.program_id(0) * bi
            + lax.broadcasted_iota(jnp.int32, (1, bi), 1))
    same = eic == eir
    beats = (epc > epr) | ((epc == epr) & (jcol < irow))
    rank = jnp.sum((same & beats).astype(jnp.int32), axis=0,
                   keepdims=True)

    e3 = lax.broadcasted_iota(jnp.int32, (_E, _CAP, bi), 0)
    c3 = lax.broadcasted_iota(jnp.int32, (_E, _CAP, bi), 1)
    hit = (e3 == eir[None, :, :]) & (c3 == rank[None, :, :])
    combine_ref[...] = jnp.where(hit, epr[None, :, :], 0.0)


def kernel(X, W_gate, current_epoch):
    n, c, h, w = X.shape
    bt = 64
    Xp = jnp.transpose(X, (2, 3, 0, 1))
    pooled = pl.pallas_call(
        _pool_kernel,
        grid=(n // bt,),
        in_specs=[
            pl.BlockSpec((h, w, bt, c), lambda i: (0, 0, i, 0)),
        ],
        out_specs=pl.BlockSpec((bt, c), lambda i: (i, 0)),
        out_shape=jax.ShapeDtypeStruct((n, c), jnp.float32),
        compiler_params=pltpu.CompilerParams(
            dimension_semantics=("parallel",)),
    )(Xp)

    logits_raw = (pooled @ W_gate).astype(jnp.float32)
    logits = jnp.clip(logits_raw, -10.0, 10.0) / _TEMP
    probs = jax.nn.softmax(logits, axis=1)
    eidx = jnp.argmax(probs, axis=1).astype(jnp.int32)
    ep = jnp.max(probs, axis=1, keepdims=True)

    scalar = jax.ShapeDtypeStruct((1, 1), jnp.float32)
    stdv, z, aux, div = pl.pallas_call(
        _stats_kernel,
        out_shape=(scalar, scalar, scalar, scalar),
    )(logits_raw.T)

    ep_row = ep.reshape(1, n)
    eidx_row = eidx.reshape(1, n)
    eidx_col = eidx.reshape(n, 1)

    bi = 256
    combine_t = pl.pallas_call(
        _route_kernel,
        grid=(n // bi,),
        in_specs=[
            pl.BlockSpec((n, 1), lambda i: (0, 0)),
            pl.BlockSpec((n, 1), lambda i: (0, 0)),
            pl.BlockSpec((1, bi), lambda i: (0, i)),
            pl.BlockSpec((1, bi), lambda i: (0, i)),
        ],
        out_specs=pl.BlockSpec((_E, _CAP, bi), lambda i: (0, 0, i)),
        out_shape=jax.ShapeDtypeStruct((_E, _CAP, n), jnp.float32),
        compiler_params=pltpu.CompilerParams(
            dimension_semantics=("arbitrary",)),
    )(ep, eidx_col, ep_row, eidx_row)

    combine = jnp.transpose(combine_t, (2, 0, 1))
    dispatch = combine > 0

    return (dispatch, combine, z[0, 0], aux[0, 0], div[0, 0],
            stdv[0, 0], logits)
